# baseline (speedup 1.0000x reference)
import functools

import jax
import jax.numpy as jnp
import numpy as np
from jax import lax
from jax.experimental import pallas as pl
from jax.experimental.pallas import tpu as pltpu
from jax.experimental.pallas import tpu_sc as plsc

NUM_VOCAB = 1000000
D_MODEL = 64
SEQ_LEN = 200
N_SEQ = 4096
SCALE = 8.0

NC, NS = 2, 16
NW = NC * NS
SEQ_PER_W = N_SEQ // NW
CS = 4
ROWS = CS * SEQ_LEN
NCH = SEQ_PER_W // CS
GSUB = 100
NG = ROWS // GSUB


def _pos_encoding() -> np.ndarray:
    position = np.arange(0, 512, dtype=np.float64)[:, None]
    div_term = np.exp(
        -np.arange(0, D_MODEL, 2, dtype=np.float64) * (np.log(10000.0) / D_MODEL)
    )
    pe = np.zeros((512, D_MODEL), dtype=np.float32)
    pe[:, 0::2] = np.sin(position * div_term)
    pe[:, 1::2] = np.cos(position * div_term)
    return pe[:SEQ_LEN]


_PE = _pos_encoding()


def _body(idx_hbm, pe_hbm, table_hbm, out_hbm, idx_v, rows_v, pe_v, sem):
    wid = lax.axis_index("s") * NC + lax.axis_index("c")
    pltpu.sync_copy(pe_hbm, pe_v)

    def chunk(c, carry):
        row0 = pl.multiple_of((wid * SEQ_PER_W + c * CS) * SEQ_LEN, ROWS)
        pltpu.sync_copy(idx_hbm.at[pl.ds(pl.multiple_of(row0 // GSUB, NG), NG)], idx_v)
        copies = [
            pltpu.async_copy(
                table_hbm.at[idx_v.at[g]], rows_v.at[pl.ds(g * GSUB, GSUB)], sem
            )
            for g in range(NG)
        ]
        for cp in copies:
            cp.wait()

        def fma(t, carry):
            for s in range(CS):
                r = s * SEQ_LEN + t
                for q in range(D_MODEL // 16):
                    sl = pl.ds(q * 16, 16)
                    rows_v[r, sl] = rows_v[r, sl] * SCALE + pe_v[t, sl]
            return carry

        lax.fori_loop(0, SEQ_LEN, fma, 0, unroll=2)
        pltpu.sync_copy(rows_v, out_hbm.at[pl.ds(row0, ROWS)])
        return carry

    lax.fori_loop(0, NCH, chunk, 0)


@functools.partial(jax.jit, static_argnames=())
def kernel(x, table):
    idx = x.astype(jnp.int32).reshape(-1, GSUB)
    pe = jnp.asarray(_PE)
    call = pl.kernel(
        _body,
        out_type=jax.ShapeDtypeStruct((N_SEQ * SEQ_LEN, D_MODEL), jnp.float32),
        mesh=plsc.VectorSubcoreMesh(core_axis_name="c", subcore_axis_name="s"),
        scratch_types=[
            pltpu.VMEM((NG, GSUB), jnp.int32),
            pltpu.VMEM((ROWS, D_MODEL), jnp.float32),
            pltpu.VMEM((SEQ_LEN, D_MODEL), jnp.float32),
            pltpu.SemaphoreType.DMA,
        ],
        compiler_params=pltpu.CompilerParams(use_tc_tiling_on_sc=False),
    )
    out = call(idx, pe, table)
    return out.reshape(N_SEQ, SEQ_LEN, D_MODEL)
